# baseline (speedup 1.0000x reference)
import functools

import jax
import jax.numpy as jnp
from jax import lax
from jax.experimental import pallas as pl
from jax.experimental.pallas import tpu as pltpu
from jax.experimental.pallas import tpu_sc as plsc

N = 10000
E = 320000
D = 128
FF = 512
SCALE = D ** (-0.5)

ROWW = 144
CHUNK = 32
NCHUNK = E // CHUNK
NCORE = 2
NSUB = 16
NW = NCORE * NSUB
HALVES = (NCHUNK + NW - 1) // NW
PAIRS = (HALVES + 1) // 2
NACC = 10240
ROWS_PER_SUB = NACC // NSUB
ZROWS = CHUNK


def _edge_body(q_hbm, kv_hbm, src_hbm, dst_hbm, out_hbm,
               sA, dA, sB, dB, ssA, ssB, qsA, kvdA, qsB, kvdB, ovA, ovB, acc,
               sem_sA, sem_dA, sem_sB, sem_dB,
               sem_qA, sem_kvA, sem_qB, sem_kvB, sem_ovA, sem_ovB):
    cid = lax.axis_index("c")
    sid = lax.axis_index("s")
    wid = sid * NCORE + cid

    zeros16 = jnp.zeros((16,), jnp.float32)

    def zrow(r, _):
        for j in range(ROWW // 16):
            ovA[r, pl.ds(j * 16, 16)] = zeros16
        return 0
    lax.fori_loop(0, ZROWS, zrow, 0)
    base_rows = sid * ROWS_PER_SUB
    for p in range(ROWS_PER_SUB // ZROWS):
        pltpu.sync_copy(ovA.at[pl.ds(0, ZROWS)],
                        acc.at[pl.ds(base_rows + p * ZROWS, ZROWS)])
    plsc.subcore_barrier()

    def cbase(k):
        c = jnp.minimum(k * NW + wid, NCHUNK - 1)
        return c * CHUNK

    e0 = (lax.iota(jnp.int32, 16) == 0).astype(jnp.float32)

    def compute(qs, kvd, ov):
        for e in range(CHUNK):
            prods = [qs[e, pl.ds(j * 16, 16)] * kvd[e, pl.ds(j * 16, 16)]
                     for j in range(D // 16)]
            while len(prods) > 1:
                prods = [a + b for a, b in zip(prods[::2], prods[1::2])]
            s = jnp.sum(prods[0]) * SCALE
            ex = jnp.exp(jnp.full((16,), s, jnp.float32))
            for j in range(D // 16):
                ov[e, pl.ds(j * 16, 16)] = ex * kvd[e, pl.ds(D + j * 16, 16)]
            ov[e, pl.ds(D, 16)] = ex * e0

    def half(i, k, s_i, d_i, ss_i, qs_i, kvd_i, ov_i,
             sem_s, sem_d, sem_q, sem_kv, sem_ov,
             s_o, d_o, qs_o, kvd_o, sem_so, sem_do, sem_qo, sem_kvo):
        c = k * NW + wid
        pltpu.make_async_copy(q_hbm.at[s_i], qs_i, sem_q).wait()
        pltpu.make_async_copy(kv_hbm.at[d_i], kvd_i, sem_kv).wait()
        @pl.when(jnp.logical_and(i > 0, (k - 2) * NW + wid < NCHUNK))
        def _():
            pltpu.make_async_copy(ov_i, acc.at[ss_i], sem_ov).wait()
        for j in range(CHUNK // 16):
            ss_i[pl.ds(j * 16, 16)] = s_i[pl.ds(j * 16, 16)]
        pltpu.async_copy(src_hbm.at[pl.ds(cbase(k + 2), CHUNK)], s_i, sem_s)
        pltpu.async_copy(dst_hbm.at[pl.ds(cbase(k + 2), CHUNK)], d_i, sem_d)
        pltpu.make_async_copy(src_hbm.at[pl.ds(cbase(k + 1), CHUNK)],
                              s_o, sem_so).wait()
        pltpu.make_async_copy(dst_hbm.at[pl.ds(cbase(k + 1), CHUNK)],
                              d_o, sem_do).wait()
        pltpu.async_copy(q_hbm.at[s_o], qs_o, sem_qo)
        pltpu.async_copy(kv_hbm.at[d_o], kvd_o, sem_kvo)
        compute(qs_i, kvd_i, ov_i)

        @pl.when(c < NCHUNK)
        def _():
            pltpu.async_copy(ov_i, acc.at[ss_i], sem_ov, add=True)

    pltpu.async_copy(src_hbm.at[pl.ds(cbase(0), CHUNK)], sA, sem_sA)
    pltpu.async_copy(dst_hbm.at[pl.ds(cbase(0), CHUNK)], dA, sem_dA)
    pltpu.async_copy(src_hbm.at[pl.ds(cbase(1), CHUNK)], sB, sem_sB)
    pltpu.async_copy(dst_hbm.at[pl.ds(cbase(1), CHUNK)], dB, sem_dB)
    pltpu.make_async_copy(src_hbm.at[pl.ds(cbase(0), CHUNK)], sA, sem_sA).wait()
    pltpu.make_async_copy(dst_hbm.at[pl.ds(cbase(0), CHUNK)], dA, sem_dA).wait()
    pltpu.async_copy(q_hbm.at[sA], qsA, sem_qA)
    pltpu.async_copy(kv_hbm.at[dA], kvdA, sem_kvA)

    def pair_body(i, _):
        half(i, 2 * i, sA, dA, ssA, qsA, kvdA, ovA,
             sem_sA, sem_dA, sem_qA, sem_kvA, sem_ovA,
             sB, dB, qsB, kvdB, sem_sB, sem_dB, sem_qB, sem_kvB)
        half(i, 2 * i + 1, sB, dB, ssB, qsB, kvdB, ovB,
             sem_sB, sem_dB, sem_qB, sem_kvB, sem_ovB,
             sA, dA, qsA, kvdA, sem_sA, sem_dA, sem_qA, sem_kvA)
        return 0
    lax.fori_loop(0, PAIRS, pair_body, 0)

    pltpu.make_async_copy(q_hbm.at[sA], qsA, sem_qA).wait()
    pltpu.make_async_copy(kv_hbm.at[dA], kvdA, sem_kvA).wait()
    pltpu.make_async_copy(src_hbm.at[pl.ds(0, CHUNK)], sB, sem_sB).wait()
    pltpu.make_async_copy(dst_hbm.at[pl.ds(0, CHUNK)], dB, sem_dB).wait()
    @pl.when((2 * PAIRS - 2) * NW + wid < NCHUNK)
    def _():
        pltpu.make_async_copy(ovA, acc.at[ssA], sem_ovA).wait()

    @pl.when((2 * PAIRS - 1) * NW + wid < NCHUNK)
    def _():
        pltpu.make_async_copy(ovB, acc.at[ssB], sem_ovB).wait()

    plsc.subcore_barrier()
    pltpu.sync_copy(acc.at[pl.ds(base_rows, ROWS_PER_SUB)],
                    out_hbm.at[cid, pl.ds(base_rows, ROWS_PER_SUB)])


@functools.cache
def _edge_call():
    return pl.kernel(
        _edge_body,
        mesh=plsc.VectorSubcoreMesh(core_axis_name="c", subcore_axis_name="s"),
        out_type=jax.ShapeDtypeStruct((NCORE, NACC, ROWW), jnp.float32),
        compiler_params=pltpu.CompilerParams(use_tc_tiling_on_sc=False, needs_layout_passes=False),
        scratch_types=[
            pltpu.VMEM((CHUNK,), jnp.int32),
            pltpu.VMEM((CHUNK,), jnp.int32),
            pltpu.VMEM((CHUNK,), jnp.int32),
            pltpu.VMEM((CHUNK,), jnp.int32),
            pltpu.VMEM((CHUNK,), jnp.int32),
            pltpu.VMEM((CHUNK,), jnp.int32),
            pltpu.VMEM((CHUNK, D), jnp.float32),
            pltpu.VMEM((CHUNK, 2 * D), jnp.float32),
            pltpu.VMEM((CHUNK, D), jnp.float32),
            pltpu.VMEM((CHUNK, 2 * D), jnp.float32),
            pltpu.VMEM((CHUNK, ROWW), jnp.float32),
            pltpu.VMEM((CHUNK, ROWW), jnp.float32),
            pltpu.VMEM_SHARED((NACC, ROWW), jnp.float32),
            pltpu.SemaphoreType.DMA,
            pltpu.SemaphoreType.DMA,
            pltpu.SemaphoreType.DMA,
            pltpu.SemaphoreType.DMA,
            pltpu.SemaphoreType.DMA,
            pltpu.SemaphoreType.DMA,
            pltpu.SemaphoreType.DMA,
            pltpu.SemaphoreType.DMA,
            pltpu.SemaphoreType.DMA,
            pltpu.SemaphoreType.DMA,
        ],
    )


BQ = 400


def _qkv_body(x_ref, w_ref, b_ref, q_ref, kv_ref):
    xb = x_ref[...]
    qkv = jnp.dot(xb, w_ref[...].T, preferred_element_type=jnp.float32)
    qkv = qkv + b_ref[...]
    q_ref[...] = qkv[:, :D]
    kv_ref[...] = qkv[:, D:]


_qkv_call = pl.pallas_call(
    _qkv_body,
    grid=(N // BQ,),
    in_specs=[
        pl.BlockSpec((BQ, D), lambda i: (i, 0)),
        pl.BlockSpec((3 * D, D), lambda i: (0, 0)),
        pl.BlockSpec((1, 3 * D), lambda i: (0, 0)),
    ],
    out_specs=[
        pl.BlockSpec((BQ, D), lambda i: (i, 0)),
        pl.BlockSpec((BQ, 2 * D), lambda i: (i, 0)),
    ],
    out_shape=[
        jax.ShapeDtypeStruct((N, D), jnp.float32),
        jax.ShapeDtypeStruct((N, 2 * D), jnp.float32),
    ],
)


def _ln(h, g, b):
    mu = jnp.mean(h, axis=-1, keepdims=True)
    var = jnp.mean((h - mu) ** 2, axis=-1, keepdims=True)
    return (h - mu) * lax.rsqrt(var + 1e-5) * g + b


def _tail_body(x_ref, p_ref, w1_ref, b1_ref, w2_ref, b2_ref,
               g1_ref, be1_ref, g2_ref, be2_ref, o_ref):
    x = x_ref[...]
    p = p_ref[...]
    num = p[0, :, :D] + p[1, :, :D]
    den = p[0, :, D] + p[1, :, D]
    den = jnp.where(den == 0.0, 1.0, den)
    attn = num / den[:, None]
    h = _ln(x + attn, g1_ref[...], be1_ref[...])
    ff = jnp.maximum(
        jnp.dot(h, w1_ref[...].T, preferred_element_type=jnp.float32)
        + b1_ref[...], 0.0)
    ff = jnp.dot(ff, w2_ref[...].T, preferred_element_type=jnp.float32)
    ff = ff + b2_ref[...]
    o_ref[...] = _ln(h + ff, g2_ref[...], be2_ref[...])


_tail_call = pl.pallas_call(
    _tail_body,
    grid=(N // BQ,),
    in_specs=[
        pl.BlockSpec((BQ, D), lambda i: (i, 0)),
        pl.BlockSpec((NCORE, BQ, ROWW), lambda i: (0, i, 0)),
        pl.BlockSpec((FF, D), lambda i: (0, 0)),
        pl.BlockSpec((1, FF), lambda i: (0, 0)),
        pl.BlockSpec((D, FF), lambda i: (0, 0)),
        pl.BlockSpec((1, D), lambda i: (0, 0)),
        pl.BlockSpec((1, D), lambda i: (0, 0)),
        pl.BlockSpec((1, D), lambda i: (0, 0)),
        pl.BlockSpec((1, D), lambda i: (0, 0)),
        pl.BlockSpec((1, D), lambda i: (0, 0)),
    ],
    out_specs=pl.BlockSpec((BQ, D), lambda i: (i, 0)),
    out_shape=jax.ShapeDtypeStruct((N, D), jnp.float32),
)


def kernel(x, edge_indices, W_qkv, b_qkv, W1, b1, W2, b2, g1, beta1, g2, beta2):
    q, kv = _qkv_call(x, W_qkv, b_qkv.reshape(1, -1))
    partial = _edge_call()(q, kv, edge_indices[0], edge_indices[1])
    out = _tail_call(x, partial, W1, b1.reshape(1, -1), W2, b2.reshape(1, -1),
                     g1.reshape(1, -1), beta1.reshape(1, -1),
                     g2.reshape(1, -1), beta2.reshape(1, -1))
    return out
